# baseline (speedup 1.0000x reference)
import functools

import numpy as np
import jax
import jax.numpy as jnp
from jax import lax
from jax.experimental import pallas as pl
from jax.experimental.pallas import tpu as pltpu
from jax.experimental.pallas import tpu_sc as plsc

_VOCAB = 100000
_D = 768
_S = 2048
_B = 32
_NC = 2
_NS = 16
_NW = _NC * _NS
_PCHUNK = _S // _NW
_BLK = _PCHUNK // 2
_LANES = 16
_CVEC = _D // _LANES


def _pe_table() -> np.ndarray:
    even_i = np.arange(0, _D, 2, dtype=np.float32)
    denominator = np.power(np.float32(10000.0), even_i / np.float32(_D))
    position = np.arange(_S, dtype=np.float32).reshape(_S, 1)
    even_pe = np.sin(position / denominator)
    odd_pe = np.cos(position / denominator)
    pe = np.stack([even_pe, odd_pe], axis=2).reshape(_S, _D)
    return pe.astype(np.float32)


_PE = _pe_table()

_MESH = plsc.VectorSubcoreMesh(core_axis_name="c", subcore_axis_name="s")


@functools.partial(
    pl.kernel,
    out_type=jax.ShapeDtypeStruct((_B, _S, _D), jnp.float32),
    mesh=_MESH,
    scratch_types=[
        pltpu.VMEM((_BLK,), jnp.int32),
        pltpu.VMEM((_BLK,), jnp.int32),
        pltpu.VMEM((_PCHUNK, _D), jnp.float32),
        pltpu.VMEM((_BLK, _D), jnp.float32),
        pltpu.VMEM((_BLK, _D), jnp.float32),
        pltpu.SemaphoreType.DMA,
        pltpu.SemaphoreType.DMA,
        pltpu.SemaphoreType.DMA,
        pltpu.SemaphoreType.DMA,
    ],
)
def _embed(tokens_hbm, table_hbm, pe_hbm, out_hbm,
           idx0, idx1, pe_v, rows0, rows1, isem0, isem1, gsem0, gsem1):
    wid = lax.axis_index("s") * _NC + lax.axis_index("c")
    p0 = wid * _PCHUNK
    pltpu.sync_copy(pe_hbm.at[pl.ds(p0, _PCHUNK)], pe_v)

    def _idx_start(b, half, idx, isem):
        return pltpu.async_copy(
            tokens_hbm.at[b, pl.ds(p0 + half * _BLK, _BLK)], idx, isem)

    def _idx_wait(idx, isem):
        pltpu.make_async_copy(tokens_hbm.at[0, pl.ds(p0, _BLK)], idx, isem).wait()

    def _gather_start(idx, rows, gsem):
        return pltpu.async_copy(table_hbm.at[idx], rows, gsem)

    def _gather_wait(idx, rows, gsem):
        pltpu.make_async_copy(table_hbm.at[idx], rows, gsem).wait()

    def _add_and_store(b, half, rows):
        def row_body(r, carry):
            for c in range(_CVEC):
                sl = pl.ds(c * _LANES, _LANES)
                plsc.addupdate(rows.at[r, sl], pe_v[half * _BLK + r, sl])
            return carry

        lax.fori_loop(0, _BLK, row_body, 0)
        pltpu.sync_copy(rows, out_hbm.at[b, pl.ds(p0 + half * _BLK, _BLK)])

    _idx_start(0, 0, idx0, isem0)
    _idx_start(0, 1, idx1, isem1)
    _idx_wait(idx0, isem0)
    _gather_start(idx0, rows0, gsem0)

    def batch_body(b, carry):
        bn = lax.min(b + 1, _B - 1)
        _gather_wait(idx0, rows0, gsem0)
        _idx_wait(idx1, isem1)
        _gather_start(idx1, rows1, gsem1)
        _idx_start(bn, 0, idx0, isem0)
        _add_and_store(b, 0, rows0)
        _gather_wait(idx1, rows1, gsem1)
        _idx_wait(idx0, isem0)
        _gather_start(idx0, rows0, gsem0)
        _idx_start(bn, 1, idx1, isem1)
        _add_and_store(b, 1, rows1)
        return carry

    lax.fori_loop(0, _B, batch_body, 0)
    _gather_wait(idx0, rows0, gsem0)
    _idx_wait(idx1, isem1)


def kernel(tokens, table):
    return _embed(tokens, table, jnp.asarray(_PE))


# speedup vs baseline: 1.1451x; 1.1451x over previous
import functools

import numpy as np
import jax
import jax.numpy as jnp
from jax import lax
from jax.experimental import pallas as pl
from jax.experimental.pallas import tpu as pltpu
from jax.experimental.pallas import tpu_sc as plsc

_VOCAB = 100000
_D = 768
_S = 2048
_B = 32
_NC = 2
_NS = 16
_NW = _NC * _NS
_PCHUNK = _S // _NW
_NSLOT = 4
_BLK = _PCHUNK // _NSLOT
_NBLK = _B * _NSLOT
_LANES = 16
_CVEC = _D // _LANES
_UVEC = 24
_NUNIT = _BLK * _CVEC // _UVEC


def _pe_table() -> np.ndarray:
    even_i = np.arange(0, _D, 2, dtype=np.float32)
    denominator = np.power(np.float32(10000.0), even_i / np.float32(_D))
    position = np.arange(_S, dtype=np.float32).reshape(_S, 1)
    even_pe = np.sin(position / denominator)
    odd_pe = np.cos(position / denominator)
    pe = np.stack([even_pe, odd_pe], axis=2).reshape(_S, _D)
    return pe.astype(np.float32)


_PE = _pe_table()

_MESH = plsc.VectorSubcoreMesh(core_axis_name="c", subcore_axis_name="s")


@functools.partial(
    pl.kernel,
    out_type=jax.ShapeDtypeStruct((_B, _S, _D), jnp.float32),
    mesh=_MESH,
    scratch_types=[
        [pltpu.VMEM((_BLK,), jnp.int32) for _ in range(_NSLOT)],
        pltpu.VMEM((_PCHUNK, _D), jnp.float32),
        [pltpu.VMEM((_BLK, _D), jnp.float32) for _ in range(_NSLOT)],
        pltpu.SemaphoreType.DMA,
        pltpu.SemaphoreType.DMA,
        pltpu.SemaphoreType.DMA,
    ],
)
def _embed(tokens_hbm, table_hbm, pe_hbm, out_hbm,
           idx, pe_v, rows, isem, gsem, ssem):
    wid = lax.axis_index("s") * _NC + lax.axis_index("c")
    p0 = wid * _PCHUNK
    pltpu.sync_copy(pe_hbm.at[pl.ds(p0, _PCHUNK)], pe_v)

    def _tok_src(b, q):
        return tokens_hbm.at[b, pl.ds(p0 + q * _BLK, _BLK)]

    def _idx_start(b, q, s):
        pltpu.async_copy(_tok_src(b, q), idx[s], isem)

    def _idx_wait(s):
        pltpu.make_async_copy(_tok_src(0, 0), idx[s], isem).wait()

    def _gather_start(s):
        pltpu.async_copy(table_hbm.at[idx[s]], rows[s], gsem)

    def _gather_wait(s):
        pltpu.make_async_copy(table_hbm.at[idx[s]], rows[s], gsem).wait()

    def _store_start(b, q, s):
        pltpu.async_copy(rows[s], out_hbm.at[b, pl.ds(p0 + q * _BLK, _BLK)], ssem)

    def _store_wait(b, q, s):
        pltpu.make_async_copy(
            rows[s], out_hbm.at[b, pl.ds(p0 + q * _BLK, _BLK)], ssem).wait()

    def _add_block(q, s):
        def row_body(r, carry):
            for c in range(_CVEC):
                sl = pl.ds(c * _LANES, _LANES)
                plsc.addupdate(rows[s].at[r, sl], pe_v[q * _BLK + r, sl])
            return carry

        lax.fori_loop(0, _BLK, row_body, 0)

    _idx_start(0, 0, 0)
    _idx_start(0, 1, 1)
    _idx_wait(0)
    _gather_start(0)

    def batch_body(b, carry):
        bc = lax.min(b + 1, _B - 1)
        for q in range(_NSLOT):
            s = q
            sn = (q + 1) % _NSLOT
            sp = (q + 2) % _NSLOT
            bn, qn = (b, q + 1) if q + 1 < _NSLOT else (bc, 0)
            bp, qp = (b, q + 2) if q + 2 < _NSLOT else (bc, (q + 2) % _NSLOT)
            _gather_wait(s)
            _idx_wait(sn)
            if q == _NSLOT - 1:
                _store_wait(b, 0, sn)
            else:
                @pl.when(b > 0)
                def _():
                    _store_wait(b - 1, q + 1, sn)
            _gather_start(sn)
            _idx_start(bp, qp, sp)
            _add_block(q, s)
            _store_start(b, q, s)
        return carry

    lax.fori_loop(0, _B, batch_body, 0)
    _gather_wait(0)
    _idx_wait(1)
    for q in range(1, _NSLOT):
        _store_wait(_B - 1, q, q)


def kernel(tokens, table):
    return _embed(tokens, table, jnp.asarray(_PE))


# speedup vs baseline: 1.3370x; 1.1675x over previous
import functools

import numpy as np
import jax
import jax.numpy as jnp
from jax import lax
from jax.experimental import pallas as pl
from jax.experimental.pallas import tpu as pltpu
from jax.experimental.pallas import tpu_sc as plsc

_VOCAB = 100000
_D = 768
_S = 2048
_B = 32
_NC = 2
_NS = 16
_NW = _NC * _NS
_PCHUNK = _S // _NW
_PP = 8
_PWN = _PCHUNK // _PP
_GB = 4
_BGN = _B // _GB
_NSLOT = 4
_LANES = 16
_CVEC = _D // _LANES


def _pe_table() -> np.ndarray:
    even_i = np.arange(0, _D, 2, dtype=np.float32)
    denominator = np.power(np.float32(10000.0), even_i / np.float32(_D))
    position = np.arange(_S, dtype=np.float32).reshape(_S, 1)
    even_pe = np.sin(position / denominator)
    odd_pe = np.cos(position / denominator)
    pe = np.stack([even_pe, odd_pe], axis=2).reshape(_S, _D)
    return pe.astype(np.float32)


_PE = _pe_table()

_MESH = plsc.VectorSubcoreMesh(core_axis_name="c", subcore_axis_name="s")


@functools.partial(
    pl.kernel,
    out_type=jax.ShapeDtypeStruct((_B, _S, _D), jnp.float32),
    mesh=_MESH,
    scratch_types=[
        pltpu.VMEM((_PCHUNK, _B), jnp.int32),
        pltpu.VMEM((_PP, _D), jnp.float32),
        [pltpu.VMEM((_PP, _GB, _D), jnp.float32) for _ in range(_NSLOT)],
        pltpu.SemaphoreType.DMA,
        pltpu.SemaphoreType.DMA,
    ],
)
def _embed(tokens_t_hbm, table_hbm, pe_hbm, out_hbm,
           staged, pe_v, rows, gsem, ssem):
    wid = lax.axis_index("s") * _NC + lax.axis_index("c")
    p0 = wid * _PCHUNK
    pltpu.sync_copy(tokens_t_hbm.at[pl.ds(p0, _PCHUNK), :], staged)

    def _idx(pw, pp, bg):
        return staged.at[pw * _PP + pp, pl.ds(bg * _GB, _GB)]

    def _g_start(pw, bg, s):
        for pp in range(_PP):
            pltpu.async_copy(table_hbm.at[_idx(pw, pp, bg)], rows[s].at[pp],
                             gsem)

    def _g_wait(pw, bg, s):
        for pp in range(_PP):
            pltpu.make_async_copy(table_hbm.at[_idx(pw, pp, bg)],
                                  rows[s].at[pp], gsem).wait()

    def _out_dst(pw, bg, j):
        return out_hbm.at[bg * _GB + j, pl.ds(p0 + pw * _PP, _PP)]

    def _s_start(pw, bg, s):
        for j in range(_GB):
            pltpu.async_copy(rows[s].at[:, j], _out_dst(pw, bg, j), ssem)

    def _s_wait(pw, bg, s):
        for j in range(_GB):
            pltpu.make_async_copy(rows[s].at[:, j], _out_dst(pw, bg, j),
                                  ssem).wait()

    def _add(s):
        def ppbody(pp, carry):
            for c in range(_CVEC):
                sl = pl.ds(c * _LANES, _LANES)
                v = pe_v[pp, sl]
                for j in range(_GB):
                    plsc.addupdate(rows[s].at[pp, j, sl], v)
            return carry

        lax.fori_loop(0, _PP, ppbody, 0)

    _g_start(0, 0, 0)

    def pw_body(pw, carry):
        pwc = lax.min(pw + 1, _PWN - 1)
        pltpu.sync_copy(pe_hbm.at[pl.ds(p0 + pw * _PP, _PP)], pe_v)
        for bg in range(_BGN):
            s = bg % _NSLOT
            sn = (bg + 1) % _NSLOT
            bgn = (bg + 1) % _BGN
            pwn = pw if bg + 1 < _BGN else pwc
            _g_wait(pw, bg, s)
            if bg >= 3:
                _s_wait(pw, bg - 3, sn)
            else:
                @pl.when(pw > 0)
                def _():
                    _s_wait(pw - 1, bg + _BGN - 3, sn)
            _g_start(pwn, bgn, sn)
            _add(s)
            _s_start(pw, bg, s)
        return carry

    lax.fori_loop(0, _PWN, pw_body, 0)
    _g_wait(_PWN - 1, 0, 0)
    for bg in range(_BGN - 3, _BGN):
        _s_wait(_PWN - 1, bg, bg % _NSLOT)


def kernel(tokens, table):
    return _embed(jnp.transpose(tokens), table, jnp.asarray(_PE))


# speedup vs baseline: 1.6109x; 1.2049x over previous
import functools

import numpy as np
import jax
import jax.numpy as jnp
from jax import lax
from jax.experimental import pallas as pl
from jax.experimental.pallas import tpu as pltpu
from jax.experimental.pallas import tpu_sc as plsc

_VOCAB = 100000
_D = 768
_S = 2048
_B = 32
_NC = 2
_NS = 16
_NW = _NC * _NS
_PCHUNK = _S // _NW
_PP = 8
_PWN = _PCHUNK // _PP
_GB = 4
_BGN = _B // _GB
_NSLOT = 4
_LANES = 16
_CVEC = _D // _LANES


def _pe_table() -> np.ndarray:
    even_i = np.arange(0, _D, 2, dtype=np.float32)
    denominator = np.power(np.float32(10000.0), even_i / np.float32(_D))
    position = np.arange(_S, dtype=np.float32).reshape(_S, 1)
    even_pe = np.sin(position / denominator)
    odd_pe = np.cos(position / denominator)
    pe = np.stack([even_pe, odd_pe], axis=2).reshape(_S, _D)
    return pe.astype(np.float32)


_PE = _pe_table()

_MESH = plsc.VectorSubcoreMesh(core_axis_name="c", subcore_axis_name="s")


@functools.partial(
    pl.kernel,
    out_type=jax.ShapeDtypeStruct((_B, _S, _D), jnp.float32),
    mesh=_MESH,
    scratch_types=[
        pltpu.VMEM((_PCHUNK, _B), jnp.int32),
        pltpu.VMEM((_PP, _D), jnp.float32),
        [pltpu.VMEM((_PP, _GB, _D), jnp.float32) for _ in range(_NSLOT)],
        pltpu.SemaphoreType.DMA,
        pltpu.SemaphoreType.DMA,
    ],
)
def _embed(tokens_t_hbm, table_hbm, pe_hbm, out_hbm,
           staged, pe_v, rows, gsem, ssem):
    wid = lax.axis_index("s") * _NC + lax.axis_index("c")
    p0 = wid * _PCHUNK
    pltpu.sync_copy(tokens_t_hbm.at[pl.ds(p0, _PCHUNK), :], staged)

    def _idx(pw, pp, bg):
        return staged.at[pw * _PP + pp, pl.ds(bg * _GB, _GB)]

    def _g_start(pw, bg, s):
        for pp in range(_PP):
            pltpu.async_copy(table_hbm.at[_idx(pw, pp, bg)], rows[s].at[pp],
                             gsem)

    def _g_wait(pw, bg, s):
        for pp in range(_PP):
            pltpu.make_async_copy(table_hbm.at[_idx(pw, pp, bg)],
                                  rows[s].at[pp], gsem).wait()

    def _out_dst(pw, bg, j):
        return out_hbm.at[bg * _GB + j, pl.ds(p0 + pw * _PP, _PP)]

    def _s_start(pw, bg, s):
        for j in range(_GB):
            pltpu.async_copy(rows[s].at[:, j], _out_dst(pw, bg, j), ssem)

    def _s_wait(pw, bg, s):
        for j in range(_GB):
            pltpu.make_async_copy(rows[s].at[:, j], _out_dst(pw, bg, j),
                                  ssem).wait()

    def _add(s):
        def ppbody(pp, carry):
            for g in range(_CVEC // 8):
                sls = [pl.ds((g * 8 + k) * _LANES, _LANES) for k in range(8)]
                vs = [pe_v[pp, sl] for sl in sls]
                for k in range(8):
                    for j in range(_GB):
                        plsc.addupdate(rows[s].at[pp, j, sls[k]], vs[k])
            return carry

        lax.fori_loop(0, _PP, ppbody, 0)

    _g_start(0, 0, 0)

    def pw_body(pw, carry):
        pwc = lax.min(pw + 1, _PWN - 1)
        pltpu.sync_copy(pe_hbm.at[pl.ds(p0 + pw * _PP, _PP)], pe_v)
        for bg in range(_BGN):
            s = bg % _NSLOT
            sn = (bg + 1) % _NSLOT
            bgn = (bg + 1) % _BGN
            pwn = pw if bg + 1 < _BGN else pwc
            _g_wait(pw, bg, s)
            if bg >= 3:
                _s_wait(pw, bg - 3, sn)
            else:
                @pl.when(pw > 0)
                def _():
                    _s_wait(pw - 1, bg + _BGN - 3, sn)
            _g_start(pwn, bgn, sn)
            _add(s)
            _s_start(pw, bg, s)
        return carry

    lax.fori_loop(0, _PWN, pw_body, 0)
    _g_wait(_PWN - 1, 0, 0)
    for bg in range(_BGN - 3, _BGN):
        _s_wait(_PWN - 1, bg, bg % _NSLOT)


def kernel(tokens, table):
    return _embed(jnp.transpose(tokens), table, jnp.asarray(_PE))


# speedup vs baseline: 1.6651x; 1.0336x over previous
import functools

import numpy as np
import jax
import jax.numpy as jnp
from jax import lax
from jax.experimental import pallas as pl
from jax.experimental.pallas import tpu as pltpu
from jax.experimental.pallas import tpu_sc as plsc

_VOCAB = 100000
_D = 768
_S = 2048
_B = 32
_NC = 2
_NS = 16
_NW = _NC * _NS
_PCHUNK = _S // _NW
_PP = 8
_PWN = _PCHUNK // _PP
_GB = 4
_BGN = _B // _GB
_NSLOT = 4
_LANES = 16
_CVEC = _D // _LANES


def _pe_table() -> np.ndarray:
    even_i = np.arange(0, _D, 2, dtype=np.float32)
    denominator = np.power(np.float32(10000.0), even_i / np.float32(_D))
    position = np.arange(_S, dtype=np.float32).reshape(_S, 1)
    even_pe = np.sin(position / denominator)
    odd_pe = np.cos(position / denominator)
    pe = np.stack([even_pe, odd_pe], axis=2).reshape(_S, _D)
    return pe.astype(np.float32)


_PE = _pe_table()

_MESH = plsc.VectorSubcoreMesh(core_axis_name="c", subcore_axis_name="s")


@functools.partial(
    pl.kernel,
    out_type=jax.ShapeDtypeStruct((_B, _S, _D), jnp.float32),
    mesh=_MESH,
    scratch_types=[
        pltpu.VMEM((_PCHUNK, _B), jnp.int32),
        pltpu.VMEM((2, _PP, _D), jnp.float32),
        [pltpu.VMEM((_PP, _GB, _D), jnp.float32) for _ in range(_NSLOT)],
        pltpu.SemaphoreType.DMA,
        pltpu.SemaphoreType.DMA,
        pltpu.SemaphoreType.DMA,
    ],
)
def _embed(tokens_t_hbm, table_hbm, pe_hbm, out_hbm,
           staged, pe_p, rows, psem, gsem, ssem):
    wid = lax.axis_index("s") * _NC + lax.axis_index("c")
    p0 = wid * _PCHUNK
    pltpu.sync_copy(tokens_t_hbm.at[pl.ds(p0, _PCHUNK), :], staged)

    def _idx(pw, pp, bg):
        return staged.at[pw * _PP + pp, pl.ds(bg * _GB, _GB)]

    def _g_start(pw, bg, s):
        for pp in range(_PP):
            pltpu.async_copy(table_hbm.at[_idx(pw, pp, bg)], rows[s].at[pp],
                             gsem)

    def _g_wait(pw, bg, s):
        for pp in range(_PP):
            pltpu.make_async_copy(table_hbm.at[_idx(pw, pp, bg)],
                                  rows[s].at[pp], gsem).wait()

    def _out_dst(pw, bg, j):
        return out_hbm.at[bg * _GB + j, pl.ds(p0 + pw * _PP, _PP)]

    def _s_start(pw, bg, s):
        for j in range(_GB):
            pltpu.async_copy(rows[s].at[:, j], _out_dst(pw, bg, j), ssem)

    def _s_wait(pw, bg, s):
        for j in range(_GB):
            pltpu.make_async_copy(rows[s].at[:, j], _out_dst(pw, bg, j),
                                  ssem).wait()

    def _pe_start(pw, sp):
        pltpu.async_copy(pe_hbm.at[pl.ds(p0 + pw * _PP, _PP)], pe_p.at[sp],
                         psem)

    def _pe_wait(sp):
        pltpu.make_async_copy(pe_hbm.at[pl.ds(p0, _PP)], pe_p.at[sp],
                              psem).wait()

    def _add(s, sp):
        def ppbody(pp, carry):
            for g in range(_CVEC // 8):
                sls = [pl.ds((g * 8 + k) * _LANES, _LANES) for k in range(8)]
                vs = [pe_p[sp, pp, sl] for sl in sls]
                for k in range(8):
                    for j in range(_GB):
                        plsc.addupdate(rows[s].at[pp, j, sls[k]], vs[k])
            return carry

        lax.fori_loop(0, _PP, ppbody, 0)

    _pe_start(0, 0)
    _g_start(0, 0, 0)

    def pw_body(pw, carry):
        sp = lax.rem(pw, 2)
        pwc = lax.min(pw + 1, _PWN - 1)
        for bg in range(_BGN):
            s = bg % _NSLOT
            sn = (bg + 1) % _NSLOT
            bgn = (bg + 1) % _BGN
            pwn = pw if bg + 1 < _BGN else pwc
            _g_wait(pw, bg, s)
            if bg == 0:
                _pe_wait(sp)
                _pe_start(pwc, 1 - sp)
            if bg >= 3:
                _s_wait(pw, bg - 3, sn)
            else:
                @pl.when(pw > 0)
                def _():
                    _s_wait(pw - 1, bg + _BGN - 3, sn)
            _g_start(pwn, bgn, sn)
            _add(s, sp)
            _s_start(pw, bg, s)
        return carry

    lax.fori_loop(0, _PWN, pw_body, 0)
    _g_wait(_PWN - 1, 0, 0)
    _pe_wait(0)
    for bg in range(_BGN - 3, _BGN):
        _s_wait(_PWN - 1, bg, bg % _NSLOT)


def kernel(tokens, table):
    return _embed(jnp.transpose(tokens), table, jnp.asarray(_PE))


# speedup vs baseline: 1.7425x; 1.0465x over previous
import functools

import numpy as np
import jax
import jax.numpy as jnp
from jax import lax
from jax.experimental import pallas as pl
from jax.experimental.pallas import tpu as pltpu
from jax.experimental.pallas import tpu_sc as plsc

_VOCAB = 100000
_D = 768
_S = 2048
_B = 32
_NC = 2
_NS = 16
_NW = _NC * _NS
_PCHUNK = _S // _NW
_PP = 8
_PWN = _PCHUNK // _PP
_GB = 4
_BGN = _B // _GB
_NSLOT = 4
_LANES = 16
_CVEC = _D // _LANES


def _pe_table() -> np.ndarray:
    even_i = np.arange(0, _D, 2, dtype=np.float32)
    denominator = np.power(np.float32(10000.0), even_i / np.float32(_D))
    position = np.arange(_S, dtype=np.float32).reshape(_S, 1)
    even_pe = np.sin(position / denominator)
    odd_pe = np.cos(position / denominator)
    pe = np.stack([even_pe, odd_pe], axis=2).reshape(_S, _D)
    return pe.astype(np.float32)


_PE = _pe_table()

_MESH = plsc.VectorSubcoreMesh(core_axis_name="c", subcore_axis_name="s")


@functools.partial(
    pl.kernel,
    out_type=jax.ShapeDtypeStruct((_B, _S, _D), jnp.float32),
    mesh=_MESH,
    scratch_types=[
        pltpu.VMEM((_PCHUNK, _B), jnp.int32),
        pltpu.VMEM((2, _PP, _D), jnp.float32),
        [pltpu.VMEM((_PP, _GB, _D), jnp.float32) for _ in range(_NSLOT)],
        pltpu.SemaphoreType.DMA,
        pltpu.SemaphoreType.DMA,
        pltpu.SemaphoreType.DMA,
    ],
)
def _embed(tokens_t_hbm, table_hbm, pe_hbm, out_hbm,
           staged, pe_p, rows, psem, gsem, ssem):
    wid = lax.axis_index("s") * _NC + lax.axis_index("c")
    p0 = wid * _PCHUNK
    pltpu.sync_copy(tokens_t_hbm.at[pl.ds(p0, _PCHUNK), :], staged)

    def _idx(pw, pp, bg):
        return staged.at[pw * _PP + pp, pl.ds(bg * _GB, _GB)]

    def _g_start(pw, bg, s):
        for pp in range(_PP):
            pltpu.async_copy(table_hbm.at[_idx(pw, pp, bg)], rows[s].at[pp],
                             gsem)

    def _g_wait(pw, bg, s):
        for pp in range(_PP):
            pltpu.make_async_copy(table_hbm.at[_idx(pw, pp, bg)],
                                  rows[s].at[pp], gsem).wait()

    def _out_dst(pw, bg, j):
        return out_hbm.at[bg * _GB + j, pl.ds(p0 + pw * _PP, _PP)]

    def _s_start(pw, bg, s):
        for j in range(_GB):
            pltpu.async_copy(rows[s].at[:, j], _out_dst(pw, bg, j), ssem)

    def _s_wait(pw, bg, s):
        for j in range(_GB):
            pltpu.make_async_copy(rows[s].at[:, j], _out_dst(pw, bg, j),
                                  ssem).wait()

    def _pe_start(pw, sp):
        pltpu.async_copy(pe_hbm.at[pl.ds(p0 + pw * _PP, _PP)], pe_p.at[sp],
                         psem)

    def _pe_wait(sp):
        pltpu.make_async_copy(pe_hbm.at[pl.ds(p0, _PP)], pe_p.at[sp],
                              psem).wait()

    def _add(s, sp):
        def ppbody(pp, carry):
            for g in range(_CVEC // 8):
                sls = [pl.ds((g * 8 + k) * _LANES, _LANES) for k in range(8)]
                vs = [pe_p[sp, pp, sl] for sl in sls]
                for k in range(8):
                    for j in range(_GB):
                        plsc.addupdate(rows[s].at[pp, j, sls[k]], vs[k])
            return carry

        lax.fori_loop(0, _PP, ppbody, 0)

    _pe_start(0, 0)
    _g_start(0, 0, 0)
    _g_start(0, 1, 1)

    def pw_body(pw, carry):
        sp = lax.rem(pw, 2)
        pwc = lax.min(pw + 1, _PWN - 1)
        for bg in range(_BGN):
            s = bg % _NSLOT
            sn = (bg + 2) % _NSLOT
            bgn = (bg + 2) % _BGN
            pwn = pw if bg + 2 < _BGN else pwc
            _g_wait(pw, bg, s)
            if bg == 0:
                _pe_wait(sp)
                _pe_start(pwc, 1 - sp)
            if bg >= 2:
                _s_wait(pw, bg - 2, sn)
            else:
                @pl.when(pw > 0)
                def _():
                    _s_wait(pw - 1, bg + _BGN - 2, sn)
            _g_start(pwn, bgn, sn)
            _add(s, sp)
            _s_start(pw, bg, s)
        return carry

    lax.fori_loop(0, _PWN, pw_body, 0)
    _g_wait(_PWN - 1, 0, 0)
    _g_wait(_PWN - 1, 1, 1)
    _pe_wait(0)
    for bg in range(_BGN - 2, _BGN):
        _s_wait(_PWN - 1, bg, bg % _NSLOT)


def kernel(tokens, table):
    return _embed(jnp.transpose(tokens), table, jnp.asarray(_PE))
